# baseline (speedup 1.0000x reference)
import functools

import jax
import jax.numpy as jnp
from jax import lax
from jax.experimental import pallas as pl
from jax.experimental.pallas import tpu as pltpu
from jax.experimental.pallas import tpu_sc as plsc

N = 32768
R = 128
KWIN = int(N * 0.05)
NB = 256
NV = N // 16
NC = 2
NS = 16
NW = NC * NS
ROWS_PER_W = R // NW
SHIFTS = (24, 16, 8, 0)
MANT = 0x7FFFFFFF


def _body(x_hbm, out_hbm, xbuf0, xbuf1, hist, histsum, kbuf,
          sem_in0, sem_in1, sem_out0, sem_out1):
    wid = lax.axis_index("s") * NC + lax.axis_index("c")
    lane = lax.iota(jnp.int32, 16)
    ones = jnp.ones((16,), jnp.int32)
    zeros_i = jnp.zeros((16,), jnp.int32)

    def hist_slot(bucket):
        return (bucket << 4) + ((bucket + lane) & 15)

    def zero_body(i, c):
        hist[pl.ds(i * 16, 16)] = zeros_i
        return c
    lax.fori_loop(0, (16 * NB) // 16, zero_body, 0)

    def key_of(xv):
        iv = lax.bitcast_convert_type(xv, jnp.int32)
        return iv ^ ((iv >> 31) & MANT)

    bufs = (xbuf0, xbuf1)
    sin = (sem_in0, sem_in1)
    sout = (sem_out0, sem_out1)
    bases = [(wid * ROWS_PER_W + r) * N for r in range(ROWS_PER_W)]

    def in_copy(r):
        return pltpu.make_async_copy(
            x_hbm.at[pl.ds(bases[r], N)], bufs[r % 2], sin[r % 2])

    def out_copy(r):
        return pltpu.make_async_copy(
            bufs[r % 2], out_hbm.at[pl.ds(bases[r], N)], sout[r % 2])

    in_copy(0).start()
    for r in range(ROWS_PER_W):
        xbuf = bufs[r % 2]
        in_copy(r).wait()
        if r + 1 < ROWS_PER_W:
            if r >= 1:
                out_copy(r - 1).wait()
            in_copy(r + 1).start()

        U = 8

        def do_scan(ncand, k_rem):
            @plsc.parallel_loop(0, NB // 16, 1, unroll=2)
            def _(c):
                h = zeros_i
                gbase = (c * 16 + lane) << 4
                for w in range(16):
                    h = h + plsc.load_gather(hist,
                                             [gbase + ((lane + w) & 15)])
                for l in range(16):
                    hist[pl.ds(c * 256 + l * 16, 16)] = zeros_i
                histsum[pl.ds(c * 16, 16)] = h

            lim = ncand - k_rem
            def scan_body(c, carry):
                cum, bcnt, cle_at, clt_at = carry
                h = histsum[pl.ds(c * 16, 16)]
                cle = plsc.cumsum(h) + cum
                clt = cle - h
                cond = clt <= lim
                bcnt = bcnt + plsc.all_reduce_population_count(cond)
                cle_at = jnp.maximum(cle_at, jnp.where(cond, cle, zeros_i))
                clt_at = jnp.maximum(clt_at, jnp.where(cond, clt, zeros_i))
                cum = jnp.max(cle)
                return (cum, bcnt, cle_at, clt_at)

            _, bcnt, cle_at, clt_at = lax.fori_loop(
                0, NB // 16, scan_body,
                (jnp.int32(0), zeros_i, zeros_i, zeros_i))
            bstar = jnp.max(bcnt) - 1
            cle_s = jnp.max(cle_at)
            clt_s = jnp.max(clt_at)
            return bstar, k_rem - (ncand - cle_s), cle_s - clt_s

        @plsc.parallel_loop(0, NV, 1, unroll=U)
        def _(j):
            key = key_of(xbuf[pl.ds(j * 16, 16)])
            bucket = (key >> 24) + 128
            plsc.addupdate_scatter(hist, [hist_slot(bucket)], ones)

        b0, k_rem, ncand = do_scan(jnp.int32(N), jnp.int32(KWIN))
        p0 = b0 - 128
        pv0 = jnp.full((16,), p0, jnp.int32)

        C = 4
        @plsc.parallel_loop(0, NV, C, unroll=2, carry=zeros_i)
        def _(j, base):
            for u in range(C):
                key = key_of(xbuf[pl.ds((j + u) * 16, 16)])
                m = (key >> 24) == pv0
                mi = jnp.where(m, ones, zeros_i)
                pos = base + (plsc.cumsum(mi) - mi)
                plsc.store_scatter(kbuf, [pos], key, mask=m)
                base = base + plsc.all_reduce_population_count(m)
            return base

        plsc.store_scatter(kbuf, [ncand + lane],
                           jnp.full((16,), (p0 ^ 1) << 24, jnp.int32))
        trips = (ncand + 15) >> 4

        @plsc.parallel_loop(0, trips, 1, unroll=2)
        def _(j):
            key = kbuf[pl.ds(j * 16, 16)]
            m = (key >> 24) == pv0
            bucket = (key >> 16) & 255
            plsc.addupdate_scatter(hist, [hist_slot(bucket)], ones, mask=m)

        b1, k_rem, ncand = do_scan(ncand, k_rem)
        p1 = (p0 << 8) | b1
        pv1 = jnp.full((16,), p1, jnp.int32)

        @plsc.parallel_loop(0, trips, 1, unroll=2)
        def _(j):
            key = kbuf[pl.ds(j * 16, 16)]
            m = (key >> 16) == pv1
            bucket = (key >> 8) & 255
            plsc.addupdate_scatter(hist, [hist_slot(bucket)], ones, mask=m)

        b2, k_rem, ncand = do_scan(ncand, k_rem)
        p2 = (p1 << 8) | b2
        pv2 = jnp.full((16,), p2, jnp.int32)

        @plsc.parallel_loop(0, trips, 1, unroll=2)
        def _(j):
            key = kbuf[pl.ds(j * 16, 16)]
            m = (key >> 8) == pv2
            bucket = key & 255
            plsc.addupdate_scatter(hist, [hist_slot(bucket)], ones, mask=m)

        b3, _, _ = do_scan(ncand, k_rem)
        p = (p2 << 8) | b3

        tbits = jnp.where(p >= 0, p, p ^ MANT)
        thresh = lax.bitcast_convert_type(jnp.full((16,), tbits, jnp.int32),
                                          jnp.float32)
        zf = jnp.zeros((16,), jnp.float32)

        @plsc.parallel_loop(0, NV, 1, unroll=U)
        def _(j):
            xv = xbuf[pl.ds(j * 16, 16)]
            xbuf[pl.ds(j * 16, 16)] = jnp.where(xv >= thresh, xv, zf)

        out_copy(r).start()

    out_copy(ROWS_PER_W - 2).wait()
    out_copy(ROWS_PER_W - 1).wait()


_kwta = functools.partial(
    pl.kernel,
    out_type=jax.ShapeDtypeStruct((R * N,), jnp.float32),
    mesh=plsc.VectorSubcoreMesh(core_axis_name="c", subcore_axis_name="s"),
    compiler_params=pltpu.CompilerParams(needs_layout_passes=False),
    scratch_types=[
        pltpu.VMEM((N,), jnp.float32),
        pltpu.VMEM((N,), jnp.float32),
        pltpu.VMEM((16 * NB,), jnp.int32),
        pltpu.VMEM((NB,), jnp.int32),
        pltpu.VMEM((N + 16,), jnp.int32),
        pltpu.SemaphoreType.DMA,
        pltpu.SemaphoreType.DMA,
        pltpu.SemaphoreType.DMA,
        pltpu.SemaphoreType.DMA,
    ],
)(_body)


def kernel(x):
    return _kwta(x.reshape(-1)).reshape(x.shape)


# speedup vs baseline: 1.6404x; 1.6404x over previous
import functools

import jax
import jax.numpy as jnp
from jax import lax
from jax.experimental import pallas as pl
from jax.experimental.pallas import tpu as pltpu
from jax.experimental.pallas import tpu_sc as plsc

N = 32768
R = 128
KWIN = int(N * 0.05)
NB = 256
NV = N // 16
NC = 2
NS = 16
NW = NC * NS
ROWS_PER_W = R // NW
SHIFTS = (24, 16, 8, 0)
MANT = 0x7FFFFFFF


def _body(x_hbm, out_hbm, xbuf0, xbuf1, hist, histsum, kbuf,
          sem_in0, sem_in1, sem_out0, sem_out1):
    wid = lax.axis_index("s") * NC + lax.axis_index("c")
    lane = lax.iota(jnp.int32, 16)
    ones = jnp.ones((16,), jnp.int32)
    zeros_i = jnp.zeros((16,), jnp.int32)

    def zero_body(i, c):
        hist[pl.ds(i * 16, 16)] = zeros_i
        return c
    lax.fori_loop(0, (16 * NB) // 16, zero_body, 0)

    def key_of(xv):
        iv = lax.bitcast_convert_type(xv, jnp.int32)
        return iv ^ ((iv >> 31) & MANT)

    bufs = (xbuf0, xbuf1)
    sin = (sem_in0, sem_in1)
    sout = (sem_out0, sem_out1)
    bases = [(wid * ROWS_PER_W + r) * N for r in range(ROWS_PER_W)]

    def in_copy(r):
        return pltpu.make_async_copy(
            x_hbm.at[pl.ds(bases[r], N)], bufs[r % 2], sin[r % 2])

    def out_copy(r):
        return pltpu.make_async_copy(
            bufs[r % 2], out_hbm.at[pl.ds(bases[r], N)], sout[r % 2])

    in_copy(0).start()
    for r in range(ROWS_PER_W):
        xbuf = bufs[r % 2]
        in_copy(r).wait()
        if r + 1 < ROWS_PER_W:
            if r >= 1:
                out_copy(r - 1).wait()
            in_copy(r + 1).start()

        U = 8

        def do_scan(ncand, k_rem):
            @plsc.parallel_loop(0, NB // 16, 1, unroll=2)
            def _(c):
                h = zeros_i
                gbase = (c * 16 + lane) << 4
                for w in range(16):
                    h = h + plsc.load_gather(hist,
                                             [gbase + ((lane + w) & 15)])
                for l in range(16):
                    hist[pl.ds(c * 256 + l * 16, 16)] = zeros_i
                histsum[pl.ds(c * 16, 16)] = h

            lim = ncand - k_rem
            def scan_body(c, carry):
                cum, bcnt, cle_at, clt_at = carry
                h = histsum[pl.ds(c * 16, 16)]
                cle = plsc.cumsum(h) + cum
                clt = cle - h
                cond = clt <= lim
                bcnt = bcnt + plsc.all_reduce_population_count(cond)
                cle_at = jnp.maximum(cle_at, jnp.where(cond, cle, zeros_i))
                clt_at = jnp.maximum(clt_at, jnp.where(cond, clt, zeros_i))
                cum = jnp.max(cle)
                return (cum, bcnt, cle_at, clt_at)

            _, bcnt, cle_at, clt_at = lax.fori_loop(
                0, NB // 16, scan_body,
                (jnp.int32(0), zeros_i, zeros_i, zeros_i))
            bstar = jnp.max(bcnt) - 1
            cle_s = jnp.max(cle_at)
            clt_s = jnp.max(clt_at)
            return bstar, k_rem - (ncand - cle_s), cle_s - clt_s

        lbase = lane

        @plsc.parallel_loop(0, NV, 1, unroll=U)
        def _(j):
            key = key_of(xbuf[pl.ds(j * 16, 16)])
            kbuf[pl.ds(j * 16, 16)] = key
            slot = ((key >> 20) & ~15) + (lbase + 2048)
            plsc.addupdate_scatter(hist, [slot], ones)

        b0, k_rem, ncand = do_scan(jnp.int32(N), jnp.int32(KWIN))
        p0 = b0 - 128
        pv0 = jnp.full((16,), p0, jnp.int32)

        @plsc.parallel_loop(0, NV, 1, unroll=U)
        def _(j):
            key = kbuf[pl.ds(j * 16, 16)]
            m = (key >> 24) == pv0
            slot = ((key >> 12) & 0xFF0) + lbase
            plsc.addupdate_scatter(hist, [slot], ones, mask=m)

        b1, k_rem, ncand = do_scan(ncand, k_rem)
        p1 = (p0 << 8) | b1
        pv1 = jnp.full((16,), p1, jnp.int32)

        @plsc.parallel_loop(0, NV, 1, unroll=U)
        def _(j):
            key = kbuf[pl.ds(j * 16, 16)]
            m = (key >> 16) == pv1
            slot = ((key >> 4) & 0xFF0) + lbase
            plsc.addupdate_scatter(hist, [slot], ones, mask=m)

        b2, k_rem, ncand = do_scan(ncand, k_rem)
        p2 = (p1 << 8) | b2
        pv2 = jnp.full((16,), p2, jnp.int32)

        @plsc.parallel_loop(0, NV, 1, unroll=U)
        def _(j):
            key = kbuf[pl.ds(j * 16, 16)]
            m = (key >> 8) == pv2
            slot = ((key << 4) & 0xFF0) + lbase
            plsc.addupdate_scatter(hist, [slot], ones, mask=m)

        b3, _, _ = do_scan(ncand, k_rem)
        p = (p2 << 8) | b3

        tbits = jnp.where(p >= 0, p, p ^ MANT)
        thresh = lax.bitcast_convert_type(jnp.full((16,), tbits, jnp.int32),
                                          jnp.float32)
        zf = jnp.zeros((16,), jnp.float32)

        @plsc.parallel_loop(0, NV, 1, unroll=U)
        def _(j):
            xv = xbuf[pl.ds(j * 16, 16)]
            xbuf[pl.ds(j * 16, 16)] = jnp.where(xv >= thresh, xv, zf)

        out_copy(r).start()

    out_copy(ROWS_PER_W - 2).wait()
    out_copy(ROWS_PER_W - 1).wait()


_kwta = functools.partial(
    pl.kernel,
    out_type=jax.ShapeDtypeStruct((R * N,), jnp.float32),
    mesh=plsc.VectorSubcoreMesh(core_axis_name="c", subcore_axis_name="s"),
    compiler_params=pltpu.CompilerParams(needs_layout_passes=False),
    scratch_types=[
        pltpu.VMEM((N,), jnp.float32),
        pltpu.VMEM((N,), jnp.float32),
        pltpu.VMEM((16 * NB,), jnp.int32),
        pltpu.VMEM((NB,), jnp.int32),
        pltpu.VMEM((N,), jnp.int32),
        pltpu.SemaphoreType.DMA,
        pltpu.SemaphoreType.DMA,
        pltpu.SemaphoreType.DMA,
        pltpu.SemaphoreType.DMA,
    ],
)(_body)


def kernel(x):
    return _kwta(x.reshape(-1)).reshape(x.shape)


# speedup vs baseline: 2.3841x; 1.4533x over previous
import functools

import jax
import jax.numpy as jnp
from jax import lax
from jax.experimental import pallas as pl
from jax.experimental.pallas import tpu as pltpu
from jax.experimental.pallas import tpu_sc as plsc

N = 32768
R = 128
KWIN = int(N * 0.05)
NB = 256
NV = N // 16
NC = 2
NS = 16
NW = NC * NS
ROWS_PER_W = R // NW
SHIFTS = (24, 16, 8, 0)
MANT = 0x7FFFFFFF


def _body(x_hbm, out_hbm, xbuf0, xbuf1, hist, histsum, kbuf,
          sem_in0, sem_in1, sem_out0, sem_out1):
    wid = lax.axis_index("s") * NC + lax.axis_index("c")
    lane = lax.iota(jnp.int32, 16)
    ones = jnp.ones((16,), jnp.int32)
    zeros_i = jnp.zeros((16,), jnp.int32)

    def zero_body(i, c):
        hist[pl.ds(i * 16, 16)] = zeros_i
        return c
    lax.fori_loop(0, (16 * NB) // 16, zero_body, 0)

    def key_of(xv):
        iv = lax.bitcast_convert_type(xv, jnp.int32)
        return iv ^ ((iv >> 31) & MANT)

    bufs = (xbuf0, xbuf1)
    sin = (sem_in0, sem_in1)
    sout = (sem_out0, sem_out1)
    def in_copy(r):
        return pltpu.make_async_copy(
            x_hbm.at[wid * ROWS_PER_W + r], bufs[r % 2], sin[r % 2])

    def out_copy(r):
        return pltpu.make_async_copy(
            bufs[r % 2], out_hbm.at[wid * ROWS_PER_W + r], sout[r % 2])

    in_copy(0).start()
    for r in range(ROWS_PER_W):
        xbuf = bufs[r % 2]
        in_copy(r).wait()
        if r + 1 < ROWS_PER_W:
            if r >= 1:
                out_copy(r - 1).wait()
            in_copy(r + 1).start()

        U = 8

        def do_scan(ncand, k_rem):
            @plsc.parallel_loop(0, NB // 16, 1, unroll=2)
            def _(c):
                h = zeros_i
                gbase = (c * 16 + lane) << 4
                for w in range(16):
                    h = h + plsc.load_gather(hist,
                                             [gbase + ((lane + w) & 15)])
                for l in range(16):
                    hist[pl.ds(c * 256 + l * 16, 16)] = zeros_i
                histsum[pl.ds(c * 16, 16)] = h

            lim = ncand - k_rem
            def scan_body(c, carry):
                cum, bcnt, cle_at, clt_at = carry
                h = histsum[pl.ds(c * 16, 16)]
                cle = plsc.cumsum(h) + cum
                clt = cle - h
                cond = clt <= lim
                bcnt = bcnt + plsc.all_reduce_population_count(cond)
                cle_at = jnp.maximum(cle_at, jnp.where(cond, cle, zeros_i))
                clt_at = jnp.maximum(clt_at, jnp.where(cond, clt, zeros_i))
                cum = jnp.max(cle)
                return (cum, bcnt, cle_at, clt_at)

            _, bcnt, cle_at, clt_at = lax.fori_loop(
                0, NB // 16, scan_body,
                (jnp.int32(0), zeros_i, zeros_i, zeros_i))
            bstar = jnp.max(bcnt) - 1
            cle_s = jnp.max(cle_at)
            clt_s = jnp.max(clt_at)
            return bstar, k_rem - (ncand - cle_s), cle_s - clt_s

        lbase = lane

        @plsc.parallel_loop(0, NV, 1, unroll=U)
        def _(j):
            key = key_of(xbuf[pl.ds(j * 16, 16)])
            kbuf[pl.ds(j * 16, 16)] = key
            slot = ((key >> 20) & ~15) + (lbase + 2048)
            plsc.addupdate_scatter(hist, [slot], ones)

        b0, k_rem, ncand = do_scan(jnp.int32(N), jnp.int32(KWIN))
        p0 = b0 - 128
        pv0 = jnp.full((16,), p0, jnp.int32)

        @plsc.parallel_loop(0, NV, 1, unroll=U)
        def _(j):
            key = kbuf[pl.ds(j * 16, 16)]
            m = (key >> 24) == pv0
            slot = ((key >> 12) & 0xFF0) + lbase
            plsc.addupdate_scatter(hist, [slot], ones, mask=m)

        b1, k_rem, ncand = do_scan(ncand, k_rem)
        p1 = (p0 << 8) | b1
        pv1 = jnp.full((16,), p1, jnp.int32)

        @plsc.parallel_loop(0, NV, 1, unroll=U)
        def _(j):
            key = kbuf[pl.ds(j * 16, 16)]
            m = (key >> 16) == pv1
            slot = ((key >> 4) & 0xFF0) + lbase
            plsc.addupdate_scatter(hist, [slot], ones, mask=m)

        b2, k_rem, ncand = do_scan(ncand, k_rem)
        p2 = (p1 << 8) | b2
        pv2 = jnp.full((16,), p2, jnp.int32)

        @plsc.parallel_loop(0, NV, 1, unroll=U)
        def _(j):
            key = kbuf[pl.ds(j * 16, 16)]
            m = (key >> 8) == pv2
            slot = ((key << 4) & 0xFF0) + lbase
            plsc.addupdate_scatter(hist, [slot], ones, mask=m)

        b3, _, _ = do_scan(ncand, k_rem)
        p = (p2 << 8) | b3

        tbits = jnp.where(p >= 0, p, p ^ MANT)
        thresh = lax.bitcast_convert_type(jnp.full((16,), tbits, jnp.int32),
                                          jnp.float32)
        zf = jnp.zeros((16,), jnp.float32)

        @plsc.parallel_loop(0, NV, 1, unroll=U)
        def _(j):
            xv = xbuf[pl.ds(j * 16, 16)]
            xbuf[pl.ds(j * 16, 16)] = jnp.where(xv >= thresh, xv, zf)

        out_copy(r).start()

    out_copy(ROWS_PER_W - 2).wait()
    out_copy(ROWS_PER_W - 1).wait()


_kwta = functools.partial(
    pl.kernel,
    out_type=jax.ShapeDtypeStruct((R, N), jnp.float32),
    mesh=plsc.VectorSubcoreMesh(core_axis_name="c", subcore_axis_name="s"),
    compiler_params=pltpu.CompilerParams(needs_layout_passes=False),
    scratch_types=[
        pltpu.VMEM((N,), jnp.float32),
        pltpu.VMEM((N,), jnp.float32),
        pltpu.VMEM((16 * NB,), jnp.int32),
        pltpu.VMEM((NB,), jnp.int32),
        pltpu.VMEM((N,), jnp.int32),
        pltpu.SemaphoreType.DMA,
        pltpu.SemaphoreType.DMA,
        pltpu.SemaphoreType.DMA,
        pltpu.SemaphoreType.DMA,
    ],
)(_body)


def kernel(x):
    return _kwta(x)


# speedup vs baseline: 2.4509x; 1.0280x over previous
import functools

import jax
import jax.numpy as jnp
from jax import lax
from jax.experimental import pallas as pl
from jax.experimental.pallas import tpu as pltpu
from jax.experimental.pallas import tpu_sc as plsc

N = 32768
R = 128
KWIN = int(N * 0.05)
NB = 256
NV = N // 16
NC = 2
NS = 16
NW = NC * NS
ROWS_PER_W = R // NW
SHIFTS = (24, 16, 8, 0)
MANT = 0x7FFFFFFF


def _body(x_hbm, out_hbm, xbuf0, xbuf1, hist, histsum, totbuf, offbuf, kbuf,
          sem_in0, sem_in1, sem_out0, sem_out1):
    wid = lax.axis_index("s") * NC + lax.axis_index("c")
    lane = lax.iota(jnp.int32, 16)
    ones = jnp.ones((16,), jnp.int32)
    zeros_i = jnp.zeros((16,), jnp.int32)

    def zero_body(i, c):
        hist[pl.ds(i * 16, 16)] = zeros_i
        return c
    lax.fori_loop(0, (16 * NB) // 16, zero_body, 0)

    def key_of(xv):
        iv = lax.bitcast_convert_type(xv, jnp.int32)
        return iv ^ ((iv >> 31) & MANT)

    bufs = (xbuf0, xbuf1)
    sin = (sem_in0, sem_in1)
    sout = (sem_out0, sem_out1)
    def in_copy(r):
        return pltpu.make_async_copy(
            x_hbm.at[wid * ROWS_PER_W + r], bufs[r % 2], sin[r % 2])

    def out_copy(r):
        return pltpu.make_async_copy(
            bufs[r % 2], out_hbm.at[wid * ROWS_PER_W + r], sout[r % 2])

    in_copy(0).start()
    for r in range(ROWS_PER_W):
        xbuf = bufs[r % 2]
        in_copy(r).wait()
        if r + 1 < ROWS_PER_W:
            if r >= 1:
                out_copy(r - 1).wait()
            in_copy(r + 1).start()

        U = 8

        def do_scan(ncand, k_rem):
            @plsc.parallel_loop(0, NB // 16, 1, unroll=2)
            def _(c):
                h = zeros_i
                gbase = (c * 16 + lane) << 4
                for w in range(16):
                    h = h + plsc.load_gather(hist,
                                             [gbase + ((lane + w) & 15)])
                for l in range(16):
                    hist[pl.ds(c * 256 + l * 16, 16)] = zeros_i
                histsum[pl.ds(c * 16, 16)] = h
                totbuf[pl.ds(c * 16, 16)] = jnp.full((16,), jnp.sum(h),
                                                     jnp.int32)

            t = plsc.load_gather(totbuf, [lane * 16])
            offbuf[pl.ds(0, 16)] = plsc.cumsum(t) - t

            lim = ncand - k_rem

            @plsc.parallel_loop(0, NB // 16, 1, unroll=2,
                                carry=(zeros_i, zeros_i, zeros_i))
            def scanres(c, carry):
                bcnt, cle_at, clt_at = carry
                off = plsc.load_gather(offbuf, [jnp.full((16,), c,
                                                         jnp.int32)])
                h = histsum[pl.ds(c * 16, 16)]
                cle = plsc.cumsum(h) + off
                clt = cle - h
                cond = clt <= lim
                bcnt = bcnt + plsc.all_reduce_population_count(cond)
                cle_at = jnp.maximum(cle_at, jnp.where(cond, cle, zeros_i))
                clt_at = jnp.maximum(clt_at, jnp.where(cond, clt, zeros_i))
                return (bcnt, cle_at, clt_at)

            bcnt, cle_at, clt_at = scanres
            bstar = jnp.max(bcnt) - 1
            cle_s = jnp.max(cle_at)
            clt_s = jnp.max(clt_at)
            return bstar, k_rem - (ncand - cle_s), cle_s - clt_s

        lbase = lane

        @plsc.parallel_loop(0, NV, 1, unroll=U)
        def _(j):
            key = key_of(xbuf[pl.ds(j * 16, 16)])
            kbuf[pl.ds(j * 16, 16)] = key
            slot = ((key >> 20) & ~15) + (lbase + 2048)
            plsc.addupdate_scatter(hist, [slot], ones)

        b0, k_rem, ncand = do_scan(jnp.int32(N), jnp.int32(KWIN))
        p0 = b0 - 128
        pv0 = jnp.full((16,), p0, jnp.int32)

        @plsc.parallel_loop(0, NV, 1, unroll=U)
        def _(j):
            key = kbuf[pl.ds(j * 16, 16)]
            m = (key >> 24) == pv0
            slot = ((key >> 12) & 0xFF0) + lbase
            plsc.addupdate_scatter(hist, [slot], ones, mask=m)

        b1, k_rem, ncand = do_scan(ncand, k_rem)
        p1 = (p0 << 8) | b1
        pv1 = jnp.full((16,), p1, jnp.int32)

        @plsc.parallel_loop(0, NV, 1, unroll=U)
        def _(j):
            key = kbuf[pl.ds(j * 16, 16)]
            m = (key >> 16) == pv1
            slot = ((key >> 4) & 0xFF0) + lbase
            plsc.addupdate_scatter(hist, [slot], ones, mask=m)

        b2, k_rem, ncand = do_scan(ncand, k_rem)
        p2 = (p1 << 8) | b2
        pv2 = jnp.full((16,), p2, jnp.int32)

        @plsc.parallel_loop(0, NV, 1, unroll=U)
        def _(j):
            key = kbuf[pl.ds(j * 16, 16)]
            m = (key >> 8) == pv2
            slot = ((key << 4) & 0xFF0) + lbase
            plsc.addupdate_scatter(hist, [slot], ones, mask=m)

        b3, _, _ = do_scan(ncand, k_rem)
        p = (p2 << 8) | b3

        tbits = jnp.where(p >= 0, p, p ^ MANT)
        thresh = lax.bitcast_convert_type(jnp.full((16,), tbits, jnp.int32),
                                          jnp.float32)
        zf = jnp.zeros((16,), jnp.float32)

        @plsc.parallel_loop(0, NV, 1, unroll=U)
        def _(j):
            xv = xbuf[pl.ds(j * 16, 16)]
            xbuf[pl.ds(j * 16, 16)] = jnp.where(xv >= thresh, xv, zf)

        out_copy(r).start()

    out_copy(ROWS_PER_W - 2).wait()
    out_copy(ROWS_PER_W - 1).wait()


_kwta = functools.partial(
    pl.kernel,
    out_type=jax.ShapeDtypeStruct((R, N), jnp.float32),
    mesh=plsc.VectorSubcoreMesh(core_axis_name="c", subcore_axis_name="s"),
    compiler_params=pltpu.CompilerParams(needs_layout_passes=False),
    scratch_types=[
        pltpu.VMEM((N,), jnp.float32),
        pltpu.VMEM((N,), jnp.float32),
        pltpu.VMEM((16 * NB,), jnp.int32),
        pltpu.VMEM((NB,), jnp.int32),
        pltpu.VMEM((NB,), jnp.int32),
        pltpu.VMEM((16,), jnp.int32),
        pltpu.VMEM((N,), jnp.int32),
        pltpu.SemaphoreType.DMA,
        pltpu.SemaphoreType.DMA,
        pltpu.SemaphoreType.DMA,
        pltpu.SemaphoreType.DMA,
    ],
)(_body)


def kernel(x):
    return _kwta(x)
